# baseline (speedup 1.0000x reference)
import jax
import jax.numpy as jnp
from jax import lax
from jax.experimental import pallas as pl
from jax.experimental.pallas import tpu as pltpu
from jax.experimental.pallas import tpu_sc as plsc

N = 100000
D = 128
NW = 32
LANES = 16
TBLK = 2048
TGRID = 35
TCN = TGRID * TBLK
SCN = N - TCN
RB = 880
EXTRA = 10
RMAX = RB + 16
CHUNK = 176
NCHUNK = RB // CHUNK
NGROUP = CHUNK // LANES
NBUF = 3
NEG = -1.0e30
PAD = -3.0e38
FPAD = 784 * 128

_mesh = plsc.VectorSubcoreMesh(core_axis_name="c", subcore_axis_name="s")


def _wid():
    return lax.axis_index("s") * 2 + lax.axis_index("c")


def _k1_body(emb, w_hbm, scores_out, buf, w_v, scores_v, semring):
    wid = _wid()
    lstart = wid * RB + 16 * jnp.minimum(wid, EXTRA)
    start = TCN + lstart
    CD = CHUNK * D

    def issue(cin):
        sel = cin % NBUF
        pltpu.async_copy(emb.at[pl.ds((start + cin * CHUNK) * D, CD)],
                         buf.at[pl.ds(sel * CD, CD)], semring.at[sel])

    def wait_for(cin):
        sel = cin % NBUF
        pltpu.make_async_copy(emb.at[pl.ds(0, CD)],
                              buf.at[pl.ds(sel * CD, CD)],
                              semring.at[sel]).wait()

    def prebody(c0, carry):
        issue(c0)
        return carry

    lax.fori_loop(0, NBUF - 1, prebody, 0)
    pltpu.sync_copy(w_hbm, w_v)

    wvs = [w_v[pl.ds(k * LANES, LANES)] for k in range(D // LANES)]
    lane = lax.iota(jnp.int32, LANES)
    half = lane < 8

    def score_group(rowbase):
        res = jnp.zeros((LANES,), jnp.float32)
        for l2 in range(LANES // 2):
            ps = []
            for l in (2 * l2, 2 * l2 + 1):
                rb = rowbase + l * D
                p = buf[pl.ds(rb, LANES)] * wvs[0]
                for k in range(1, D // LANES):
                    p = p + buf[pl.ds(rb + k * LANES, LANES)] * wvs[k]
                ps.append(p + lax.rev(p, (0,)))
            c = plsc.cumsum(jnp.where(half, ps[0], ps[1]))
            sa = c[7]
            sb = c[15] - c[7]
            res = jnp.where(lane == 2 * l2, sa, res)
            res = jnp.where(lane == 2 * l2 + 1, sb, res)
        return res

    def chunk_body(ci, carry):
        @pl.when(ci + (NBUF - 1) < NCHUNK)
        def _():
            issue(ci + (NBUF - 1))

        wait_for(ci)
        bufbase = (ci % NBUF) * CD

        def gbody(g, c2):
            off = ci * CHUNK + g * LANES
            scores_v[pl.ds(off, LANES)] = score_group(bufbase + g * LANES * D)
            return c2

        return lax.fori_loop(0, NGROUP, gbody, carry)

    lax.fori_loop(0, NCHUNK, chunk_body, 0)

    @pl.when(wid < EXTRA)
    def _():
        pltpu.sync_copy(emb.at[pl.ds((start + RB) * D, LANES * D)],
                        buf.at[pl.ds(0, LANES * D)])
        scores_v[pl.ds(RB, LANES)] = score_group(0)

    pltpu.sync_copy(scores_v.at[pl.ds(0, RB)],
                    scores_out.at[pl.ds(lstart, RB)])

    @pl.when(wid < EXTRA)
    def _():
        pltpu.sync_copy(scores_v.at[pl.ds(RB, LANES)],
                        scores_out.at[pl.ds(lstart + RB, LANES)])


_k1 = pl.kernel(
    _k1_body,
    out_type=jax.ShapeDtypeStruct((SCN,), jnp.float32),
    mesh=_mesh,
    compiler_params=pltpu.CompilerParams(needs_layout_passes=False),
    scratch_types=[
        pltpu.VMEM((NBUF * CHUNK * D,), jnp.float32),
        pltpu.VMEM((D,), jnp.float32),
        pltpu.VMEM((RMAX,), jnp.float32),
        pltpu.SemaphoreType.DMA((NBUF,)),
    ],
)


def _tck1_body(emb_ref, w_ref, out_ref):
    out_ref[...] = jnp.dot(emb_ref[...], w_ref[...],
                           preferred_element_type=jnp.float32)


_tck1 = pl.pallas_call(
    _tck1_body,
    grid=(TGRID,),
    in_specs=[
        pl.BlockSpec((TBLK, D), lambda i: (i, 0)),
        pl.BlockSpec((D, 8), lambda i: (0, 0)),
    ],
    out_specs=pl.BlockSpec((TBLK, 8), lambda i: (i, 0)),
    out_shape=jax.ShapeDtypeStruct((TCN, 8), jnp.float32),
)


def _fin_body(sil_ref, dec_ref, b_ref, sc_ref, out_ref):
    s = sc_ref[...] + b_ref[...]
    s = s + jnp.where(sil_ref[...] == 1, NEG, 0.0)
    s = s + jnp.where(dec_ref[...] == 1, NEG, 0.0)
    gmax = jnp.max(s)
    e = jnp.exp(s - gmax)
    out_ref[...] = e * (1.0 / jnp.sum(e))


_fin = pl.pallas_call(
    _fin_body,
    out_shape=jax.ShapeDtypeStruct((FPAD // 128, 128), jnp.float32),
)


@jax.jit
def kernel(embeddings, W, b, silent_np, decision, number_of_candidates):
    del number_of_candidates
    wmat8 = jnp.broadcast_to(W.reshape(D, 1), (D, 8))
    scores_tc = _tck1(embeddings, wmat8)[:, 0]
    scores_sc = _k1(embeddings.reshape(N * D), W.reshape(D))
    sc_all = jnp.concatenate([scores_tc, scores_sc])
    sc_pad = jnp.pad(sc_all, (0, FPAD - N),
                     constant_values=PAD).reshape(FPAD // 128, 128)
    silp = jnp.pad(silent_np, (0, FPAD - N)).reshape(FPAD // 128, 128)
    decp = jnp.pad(decision, (0, FPAD - N)).reshape(FPAD // 128, 128)
    b11 = b.reshape(1, 1)
    return _fin(silp, decp, b11, sc_pad).reshape(FPAD)[:N]


# speedup vs baseline: 1.5696x; 1.5696x over previous
import jax
import jax.numpy as jnp
from jax import lax
from jax.experimental import pallas as pl
from jax.experimental.pallas import tpu as pltpu
from jax.experimental.pallas import tpu_sc as plsc

N = 100000
D = 128
NW = 32
LANES = 16
TCN = 0
SCN = N
RB = 3120
EXTRA = 10
RMAX = RB + 16
CHUNK = 240
NCHUNK = RB // CHUNK
NGROUP = CHUNK // LANES
NBUF = 3
NEG = -1.0e30
PAD = -3.0e38
FPAD = 784 * 128

_mesh = plsc.VectorSubcoreMesh(core_axis_name="c", subcore_axis_name="s")


def _wid():
    return lax.axis_index("s") * 2 + lax.axis_index("c")


def _k1_body(emb, w_hbm, scores_out, buf, w_v, scores_v, semring):
    wid = _wid()
    lstart = wid * RB + 16 * jnp.minimum(wid, EXTRA)
    start = TCN + lstart
    CD = CHUNK * D

    def issue(cin):
        sel = cin % NBUF
        pltpu.async_copy(emb.at[pl.ds((start + cin * CHUNK) * D, CD)],
                         buf.at[pl.ds(sel * CD, CD)], semring.at[sel])

    def wait_for(cin):
        sel = cin % NBUF
        pltpu.make_async_copy(emb.at[pl.ds(0, CD)],
                              buf.at[pl.ds(sel * CD, CD)],
                              semring.at[sel]).wait()

    def prebody(c0, carry):
        issue(c0)
        return carry

    lax.fori_loop(0, NBUF - 1, prebody, 0)
    pltpu.sync_copy(w_hbm, w_v)

    wvs = [w_v[pl.ds(k * LANES, LANES)] for k in range(D // LANES)]
    lane = lax.iota(jnp.int32, LANES)
    half = lane < 8

    def score_group(rowbase):
        res = jnp.zeros((LANES,), jnp.float32)
        for l2 in range(LANES // 2):
            ps = []
            for l in (2 * l2, 2 * l2 + 1):
                rb = rowbase + l * D
                p = buf[pl.ds(rb, LANES)] * wvs[0]
                for k in range(1, D // LANES):
                    p = p + buf[pl.ds(rb + k * LANES, LANES)] * wvs[k]
                ps.append(p + lax.rev(p, (0,)))
            c = plsc.cumsum(jnp.where(half, ps[0], ps[1]))
            sa = c[7]
            sb = c[15] - c[7]
            res = jnp.where(lane == 2 * l2, sa, res)
            res = jnp.where(lane == 2 * l2 + 1, sb, res)
        return res

    def chunk_body(ci, carry):
        @pl.when(ci + (NBUF - 1) < NCHUNK)
        def _():
            issue(ci + (NBUF - 1))

        wait_for(ci)
        bufbase = (ci % NBUF) * CD

        def gbody(g, c2):
            off = ci * CHUNK + g * LANES
            scores_v[pl.ds(off, LANES)] = score_group(bufbase + g * LANES * D)
            return c2

        return lax.fori_loop(0, NGROUP, gbody, carry)

    lax.fori_loop(0, NCHUNK, chunk_body, 0)

    @pl.when(wid < EXTRA)
    def _():
        pltpu.sync_copy(emb.at[pl.ds((start + RB) * D, LANES * D)],
                        buf.at[pl.ds(0, LANES * D)])
        scores_v[pl.ds(RB, LANES)] = score_group(0)

    pltpu.sync_copy(scores_v.at[pl.ds(0, RB)],
                    scores_out.at[pl.ds(lstart, RB)])

    @pl.when(wid < EXTRA)
    def _():
        pltpu.sync_copy(scores_v.at[pl.ds(RB, LANES)],
                        scores_out.at[pl.ds(lstart + RB, LANES)])


_k1 = pl.kernel(
    _k1_body,
    out_type=jax.ShapeDtypeStruct((SCN,), jnp.float32),
    mesh=_mesh,
    compiler_params=pltpu.CompilerParams(needs_layout_passes=False),
    scratch_types=[
        pltpu.VMEM((NBUF * CHUNK * D,), jnp.float32),
        pltpu.VMEM((D,), jnp.float32),
        pltpu.VMEM((RMAX,), jnp.float32),
        pltpu.SemaphoreType.DMA((NBUF,)),
    ],
)


def _fin_body(sil_ref, dec_ref, b_ref, sc_ref, out_ref):
    s = sc_ref[...] + b_ref[...]
    s = s + jnp.where(sil_ref[...] == 1, NEG, 0.0)
    s = s + jnp.where(dec_ref[...] == 1, NEG, 0.0)
    gmax = jnp.max(s)
    e = jnp.exp(s - gmax)
    out_ref[...] = e * (1.0 / jnp.sum(e))


_fin = pl.pallas_call(
    _fin_body,
    out_shape=jax.ShapeDtypeStruct((FPAD // 128, 128), jnp.float32),
)


@jax.jit
def kernel(embeddings, W, b, silent_np, decision, number_of_candidates):
    del number_of_candidates
    sc_all = _k1(embeddings.reshape(N * D), W.reshape(D))
    sc_pad = jnp.pad(sc_all, (0, FPAD - N),
                     constant_values=PAD).reshape(FPAD // 128, 128)
    silp = jnp.pad(silent_np, (0, FPAD - N)).reshape(FPAD // 128, 128)
    decp = jnp.pad(decision, (0, FPAD - N)).reshape(FPAD // 128, 128)
    b11 = b.reshape(1, 1)
    return _fin(silp, decp, b11, sc_pad).reshape(FPAD)[:N]
